# baseline (speedup 1.0000x reference)
import functools

import jax
import jax.numpy as jnp
from jax import lax
from jax.experimental import pallas as pl
from jax.experimental.pallas import tpu as pltpu
from jax.experimental.pallas import tpu_sc as plsc

_TM = 48
_PAD = 128


def _router_body(x_ref, wr_ref, br_ref, dest_ref, offs_ref):
    S, DD = x_ref.shape
    E = wr_ref.shape[1]
    x = x_ref[...]
    logits = jnp.dot(x, wr_ref[...], preferred_element_type=jnp.float32)
    logits = logits + br_ref[...]
    maxv = jnp.max(logits, axis=1, keepdims=True)
    eids = lax.broadcasted_iota(jnp.int32, (S, E), 1)
    cand = jnp.where(logits == maxv, eids, E)
    assign = jnp.min(cand, axis=1, keepdims=True)
    oh = (eids == assign).astype(jnp.float32)
    GS = 128
    G = S // GS
    g_r = lax.broadcasted_iota(jnp.int32, (GS, GS), 0)
    g_c = lax.broadcasted_iota(jnp.int32, (GS, GS), 1)
    ltg = (g_c <= g_r).astype(jnp.float32)
    incls = []
    gcounts = []
    for g in range(G):
        oh_g = oh[g * GS:(g + 1) * GS, :]
        incls.append(jnp.dot(ltg, oh_g, preferred_element_type=jnp.float32))
        gcounts.append(jnp.sum(oh_g, axis=0, keepdims=True))
    gc = jnp.concatenate(gcounts, axis=0)
    q_r = lax.broadcasted_iota(jnp.int32, (G, G), 0)
    q_c = lax.broadcasted_iota(jnp.int32, (G, G), 1)
    ltq = (q_r > q_c).astype(jnp.float32)
    base = jnp.dot(ltq, gc, precision=lax.Precision.HIGHEST,
                   preferred_element_type=jnp.float32)
    counts = jnp.sum(gc, axis=0, keepdims=True)
    e_r = lax.broadcasted_iota(jnp.int32, (E, E), 0)
    e_c = lax.broadcasted_iota(jnp.int32, (E, E), 1)
    mstrict = (e_r < e_c).astype(jnp.float32)
    offs = jnp.dot(counts, mstrict, precision=lax.Precision.HIGHEST,
                   preferred_element_type=jnp.float32)
    for g in range(G):
        oh_g = oh[g * GS:(g + 1) * GS, :]
        pos_g = incls[g] + base[g:g + 1, :] + offs
        dest_g = jnp.sum(oh_g * pos_g, axis=1, keepdims=True) - 1.0
        dest_ref[g * GS:(g + 1) * GS, :] = dest_g.astype(jnp.int32)
    offs_i = offs.astype(jnp.int32)
    offs_ref[...] = jnp.concatenate(
        [offs_i, jnp.full((1, 128 - E), S, jnp.int32)], axis=1)


def _router(x2, wr, br2):
    S, DD = x2.shape
    return pl.pallas_call(
        _router_body,
        out_shape=[
            jax.ShapeDtypeStruct((S, 1), jnp.int32),
            jax.ShapeDtypeStruct((1, 128), jnp.int32),
        ],
    )(x2, wr, br2)


_NF = 2


def _ffn_body(offs_ref, x_ref, wg_ref, bg_ref, wu_ref, bu_ref, wd_ref, bd_ref,
              out_ref):
    e = pl.program_id(0)
    f = pl.program_id(1)
    start = offs_ref[0, e]
    end = offs_ref[0, e + 1]
    start_al = (start // 8) * 8
    nchunks = (end - start_al + _TM - 1) // _TM
    wg = wg_ref[0]
    wu = wu_ref[0]
    wd = wd_ref[0]
    bg = bg_ref[0]
    bu = bu_ref[0]
    bd = bd_ref[0]

    def body(j, carry):
        s = pl.multiple_of(start_al + j * _TM, 8)
        xc = x_ref[pl.ds(s, _TM), :]
        gate = jnp.dot(xc, wg, preferred_element_type=jnp.float32) + bg
        up = jnp.dot(xc, wu, preferred_element_type=jnp.float32) + bu
        h = up * (gate * jax.nn.sigmoid(gate))
        outc = jnp.dot(h, wd, preferred_element_type=jnp.float32)
        rows = s + lax.broadcasted_iota(jnp.int32, (_TM, 1), 0)
        valid = (rows >= start) & (rows < end)
        prev = out_ref[pl.ds(s, _TM), :]
        acc = jnp.where(f == 0, outc + bd, prev + outc)
        out_ref[pl.ds(s, _TM), :] = jnp.where(valid, acc, prev)
        return carry

    lax.fori_loop(0, nchunks, body, 0)


def _ffn(offs, x_sorted, Wg, bg3, Wu, bu3, Wd, bd3):
    SP, DD = x_sorted.shape
    E, _, F = Wg.shape
    FB = F // _NF
    return pl.pallas_call(
        _ffn_body,
        grid=(E, _NF),
        in_specs=[
            pl.BlockSpec(memory_space=pltpu.SMEM),
            pl.BlockSpec((SP, DD), lambda e, f: (0, 0)),
            pl.BlockSpec((1, DD, FB), lambda e, f: (e, 0, f)),
            pl.BlockSpec((1, 1, FB), lambda e, f: (e, 0, f)),
            pl.BlockSpec((1, DD, FB), lambda e, f: (e, 0, f)),
            pl.BlockSpec((1, 1, FB), lambda e, f: (e, 0, f)),
            pl.BlockSpec((1, FB, DD), lambda e, f: (e, f, 0)),
            pl.BlockSpec((1, 1, DD), lambda e, f: (e, 0, 0)),
        ],
        out_specs=pl.BlockSpec((SP, DD), lambda e, f: (0, 0)),
        out_shape=jax.ShapeDtypeStruct((SP, DD), jnp.float32),
    )(offs, x_sorted, Wg, bg3, Wu, bu3, Wd, bd3)


def _sc_gather(table, idx):
    DD = table.shape[1]
    N = idx.shape[0]
    info = plsc.get_sparse_core_info()
    nw = info.num_cores * info.num_subcores
    bpw = N // nw
    mesh = plsc.VectorSubcoreMesh(core_axis_name="c", subcore_axis_name="s")

    @functools.partial(
        pl.kernel,
        mesh=mesh,
        out_type=jax.ShapeDtypeStruct((N, DD), jnp.float32),
        scratch_types=[
            pltpu.VMEM((bpw,), jnp.int32),
            pltpu.VMEM((bpw, DD), jnp.float32),
            pltpu.SemaphoreType.DMA,
        ],
    )
    def k(table_hbm, idx_hbm, out_hbm, idx_v, rows_v, sem):
        wid = lax.axis_index("s") * info.num_cores + lax.axis_index("c")
        base = wid * bpw
        pltpu.sync_copy(idx_hbm.at[pl.ds(base, bpw)], idx_v)
        pltpu.async_copy(table_hbm.at[idx_v], rows_v, sem).wait()
        pltpu.sync_copy(rows_v, out_hbm.at[pl.ds(base, bpw)])

    return k(table, idx)


def _sc_scatter(rows, idx):
    S, DD = rows.shape
    info = plsc.get_sparse_core_info()
    nw = info.num_cores * info.num_subcores
    bpw = S // nw
    mesh = plsc.VectorSubcoreMesh(core_axis_name="c", subcore_axis_name="s")

    @functools.partial(
        pl.kernel,
        mesh=mesh,
        out_type=jax.ShapeDtypeStruct((S + _PAD, DD), jnp.float32),
        scratch_types=[
            pltpu.VMEM((bpw,), jnp.int32),
            pltpu.VMEM((bpw, DD), jnp.float32),
            pltpu.SemaphoreType.DMA,
        ],
    )
    def k(rows_hbm, idx_hbm, out_hbm, idx_v, rows_v, sem):
        wid = lax.axis_index("s") * info.num_cores + lax.axis_index("c")
        base = wid * bpw
        pltpu.sync_copy(idx_hbm.at[pl.ds(base, bpw)], idx_v)
        pltpu.sync_copy(rows_hbm.at[pl.ds(base, bpw)], rows_v)
        pltpu.async_copy(rows_v, out_hbm.at[idx_v], sem).wait()

    return k(rows, idx)


def kernel(x, Wr, br, Wg, bg, Wu, bu, Wd, bd):
    B, S, DD = x.shape
    E, _, F = Wg.shape
    x2 = x.reshape(S, DD)
    dest2, offs = _router(x2, Wr, br.reshape(1, E))
    dest = dest2.reshape(S)
    x_sorted = _sc_scatter(x2, dest)
    out_sorted = _ffn(offs, x_sorted, Wg, bg.reshape(E, 1, F), Wu,
                      bu.reshape(E, 1, F), Wd, bd.reshape(E, 1, DD))
    out = _sc_gather(out_sorted, dest)
    return out.reshape(B, S, DD)


# speedup vs baseline: 1.0192x; 1.0192x over previous
import functools

import jax
import jax.numpy as jnp
from jax import lax
from jax.experimental import pallas as pl
from jax.experimental.pallas import tpu as pltpu
from jax.experimental.pallas import tpu_sc as plsc

_TM = 64
_PAD = 128


def _router_body(x_ref, wr_ref, br_ref, dest_ref, offs_ref):
    S, DD = x_ref.shape
    E = wr_ref.shape[1]
    x = x_ref[...]
    logits = jnp.dot(x, wr_ref[...], preferred_element_type=jnp.float32)
    logits = logits + br_ref[...]
    maxv = jnp.max(logits, axis=1, keepdims=True)
    eids = lax.broadcasted_iota(jnp.int32, (S, E), 1)
    cand = jnp.where(logits == maxv, eids, E)
    assign = jnp.min(cand, axis=1, keepdims=True)
    oh = (eids == assign).astype(jnp.float32)
    GS = 128
    G = S // GS
    g_r = lax.broadcasted_iota(jnp.int32, (GS, GS), 0)
    g_c = lax.broadcasted_iota(jnp.int32, (GS, GS), 1)
    ltg = (g_c <= g_r).astype(jnp.float32)
    incls = []
    gcounts = []
    for g in range(G):
        oh_g = oh[g * GS:(g + 1) * GS, :]
        incls.append(jnp.dot(ltg, oh_g, preferred_element_type=jnp.float32))
        gcounts.append(jnp.sum(oh_g, axis=0, keepdims=True))
    gc = jnp.concatenate(gcounts, axis=0)
    q_r = lax.broadcasted_iota(jnp.int32, (G, G), 0)
    q_c = lax.broadcasted_iota(jnp.int32, (G, G), 1)
    ltq = (q_r > q_c).astype(jnp.float32)
    base = jnp.dot(ltq, gc, precision=lax.Precision.HIGHEST,
                   preferred_element_type=jnp.float32)
    counts = jnp.sum(gc, axis=0, keepdims=True)
    e_r = lax.broadcasted_iota(jnp.int32, (E, E), 0)
    e_c = lax.broadcasted_iota(jnp.int32, (E, E), 1)
    mstrict = (e_r < e_c).astype(jnp.float32)
    offs = jnp.dot(counts, mstrict, precision=lax.Precision.HIGHEST,
                   preferred_element_type=jnp.float32)
    for g in range(G):
        oh_g = oh[g * GS:(g + 1) * GS, :]
        pos_g = incls[g] + base[g:g + 1, :] + offs
        dest_g = jnp.sum(oh_g * pos_g, axis=1, keepdims=True) - 1.0
        dest_ref[g * GS:(g + 1) * GS, :] = dest_g.astype(jnp.int32)
    offs_i = offs.astype(jnp.int32)
    offs_ref[...] = jnp.concatenate(
        [offs_i, jnp.full((1, 128 - E), S, jnp.int32)], axis=1)


def _router(x2, wr, br2):
    S, DD = x2.shape
    return pl.pallas_call(
        _router_body,
        out_shape=[
            jax.ShapeDtypeStruct((S, 1), jnp.int32),
            jax.ShapeDtypeStruct((1, 128), jnp.int32),
        ],
    )(x2, wr, br2)


_NF = 2


def _ffn_body(offs_ref, x_ref, wg_ref, bg_ref, wu_ref, bu_ref, wd_ref, bd_ref,
              out_ref):
    e = pl.program_id(0)
    f = pl.program_id(1)
    start = offs_ref[0, e]
    end = offs_ref[0, e + 1]
    start_al = (start // 8) * 8
    nchunks = (end - start_al + _TM - 1) // _TM
    wg = wg_ref[0]
    wu = wu_ref[0]
    wd = wd_ref[0]
    bg = bg_ref[0]
    bu = bu_ref[0]
    bd = bd_ref[0]

    def body(j, carry):
        s = pl.multiple_of(start_al + j * _TM, 8)
        xc = x_ref[pl.ds(s, _TM), :]
        gate = jnp.dot(xc, wg, preferred_element_type=jnp.float32) + bg
        up = jnp.dot(xc, wu, preferred_element_type=jnp.float32) + bu
        h = up * (gate * jax.nn.sigmoid(gate))
        outc = jnp.dot(h, wd, preferred_element_type=jnp.float32)
        rows = s + lax.broadcasted_iota(jnp.int32, (_TM, 1), 0)
        valid = (rows >= start) & (rows < end)
        prev = out_ref[pl.ds(s, _TM), :]
        acc = jnp.where(f == 0, outc + bd, prev + outc)
        out_ref[pl.ds(s, _TM), :] = jnp.where(valid, acc, prev)
        return carry

    lax.fori_loop(0, nchunks, body, 0)


def _ffn(offs, x_sorted, Wg, bg3, Wu, bu3, Wd, bd3):
    SP, DD = x_sorted.shape
    E, _, F = Wg.shape
    FB = F // _NF
    return pl.pallas_call(
        _ffn_body,
        grid=(E, _NF),
        in_specs=[
            pl.BlockSpec(memory_space=pltpu.SMEM),
            pl.BlockSpec((SP, DD), lambda e, f: (0, 0)),
            pl.BlockSpec((1, DD, FB), lambda e, f: (e, 0, f)),
            pl.BlockSpec((1, 1, FB), lambda e, f: (e, 0, f)),
            pl.BlockSpec((1, DD, FB), lambda e, f: (e, 0, f)),
            pl.BlockSpec((1, 1, FB), lambda e, f: (e, 0, f)),
            pl.BlockSpec((1, FB, DD), lambda e, f: (e, f, 0)),
            pl.BlockSpec((1, 1, DD), lambda e, f: (e, 0, 0)),
        ],
        out_specs=pl.BlockSpec((SP, DD), lambda e, f: (0, 0)),
        out_shape=jax.ShapeDtypeStruct((SP, DD), jnp.float32),
    )(offs, x_sorted, Wg, bg3, Wu, bu3, Wd, bd3)


def _sc_gather(table, idx):
    DD = table.shape[1]
    N = idx.shape[0]
    info = plsc.get_sparse_core_info()
    nw = info.num_cores * info.num_subcores
    bpw = N // nw
    mesh = plsc.VectorSubcoreMesh(core_axis_name="c", subcore_axis_name="s")

    @functools.partial(
        pl.kernel,
        mesh=mesh,
        out_type=jax.ShapeDtypeStruct((N, DD), jnp.float32),
        scratch_types=[
            pltpu.VMEM((bpw,), jnp.int32),
            pltpu.VMEM((bpw, DD), jnp.float32),
            pltpu.SemaphoreType.DMA,
        ],
    )
    def k(table_hbm, idx_hbm, out_hbm, idx_v, rows_v, sem):
        wid = lax.axis_index("s") * info.num_cores + lax.axis_index("c")
        base = wid * bpw
        pltpu.sync_copy(idx_hbm.at[pl.ds(base, bpw)], idx_v)
        pltpu.async_copy(table_hbm.at[idx_v], rows_v, sem).wait()
        pltpu.sync_copy(rows_v, out_hbm.at[pl.ds(base, bpw)])

    return k(table, idx)


def _sc_scatter(rows, idx):
    S, DD = rows.shape
    info = plsc.get_sparse_core_info()
    nw = info.num_cores * info.num_subcores
    bpw = S // nw
    mesh = plsc.VectorSubcoreMesh(core_axis_name="c", subcore_axis_name="s")

    @functools.partial(
        pl.kernel,
        mesh=mesh,
        out_type=jax.ShapeDtypeStruct((S + _PAD, DD), jnp.float32),
        scratch_types=[
            pltpu.VMEM((bpw,), jnp.int32),
            pltpu.VMEM((bpw, DD), jnp.float32),
            pltpu.SemaphoreType.DMA,
        ],
    )
    def k(rows_hbm, idx_hbm, out_hbm, idx_v, rows_v, sem):
        wid = lax.axis_index("s") * info.num_cores + lax.axis_index("c")
        base = wid * bpw
        pltpu.sync_copy(idx_hbm.at[pl.ds(base, bpw)], idx_v)
        pltpu.sync_copy(rows_hbm.at[pl.ds(base, bpw)], rows_v)
        pltpu.async_copy(rows_v, out_hbm.at[idx_v], sem).wait()

    return k(rows, idx)


def kernel(x, Wr, br, Wg, bg, Wu, bu, Wd, bd):
    B, S, DD = x.shape
    E, _, F = Wg.shape
    x2 = x.reshape(S, DD)
    dest2, offs = _router(x2, Wr, br.reshape(1, E))
    dest = dest2.reshape(S)
    x_sorted = _sc_scatter(x2, dest)
    out_sorted = _ffn(offs, x_sorted, Wg, bg.reshape(E, 1, F), Wu,
                      bu.reshape(E, 1, F), Wd, bd.reshape(E, 1, DD))
    out = _sc_gather(out_sorted, dest)
    return out.reshape(B, S, DD)
